# Initial kernel scaffold; baseline (speedup 1.0000x reference)
import jax
import jax.numpy as jnp
from jax.experimental import pallas as pl


def kernel(xyz, feats, W1, g1, b1, W2, g2, b2):
    raise NotImplementedError("write your pallas kernel here")


# speedup vs baseline: 1.0000x; 1.0000x over previous
import functools

import jax
import jax.numpy as jnp
import numpy as np
from jax.experimental import pallas as pl

N_SAMPLE = 8192
K = 32
IN_CH = 16
MLP_CHANNELS = [16, 32]
EPS = 1e-5


def _knn(support, query, k, chunk=1024):
    sn = jnp.sum(support * support, axis=1)
    outs = []
    for i in range(0, query.shape[0], chunk):
        q = query[i:i + chunk]
        qn = jnp.sum(q * q, axis=1)
        d = qn[:, None] + sn[None, :] - 2.0 * (q @ support.T)
        _, idx = jax.lax.top_k(-d, k)
        outs.append(idx)
    return jnp.concatenate(outs, axis=0)


def _shared_mlp(x, W, gamma, beta):
    y = jnp.einsum('oi,gik->gok', W, x)
    mean = jnp.mean(y, axis=(0, 2), keepdims=True)
    var = jnp.mean((y - mean) ** 2, axis=(0, 2), keepdims=True)
    y = (y - mean) / jnp.sqrt(var + EPS)
    y = y * gamma[None, :, None] + beta[None, :, None]
    return jax.nn.relu(y)


def kernel(xyz, feats, W1, g1, b1, W2, g2, b2):
    B, C, N = feats.shape
    S = min(N_SAMPLE, N)
    perm = jax.random.permutation(jax.random.key(42), N)[:S]
    new_xyz = xyz[:, :, perm]
    support = xyz[0].T
    query = new_xyz[0].T
    idx = _knn(support, query, K)
    idx_t = idx.astype(jnp.int32).reshape(-1)
    gathered = feats[:, :, idx_t].reshape(B, C, S, K)
    rel_xyz = xyz[:, :, idx_t].reshape(B, 3, S, K) - new_xyz[:, :, :, None]
    group = jnp.concatenate([gathered, rel_xyz], axis=1)
    group = jnp.transpose(group, (0, 2, 1, 3)).reshape(B * S, C + 3, K)
    h = _shared_mlp(group, W1, g1, b1)
    h = _shared_mlp(h, W2, g2, b2)
    new_feats = jnp.max(h, axis=2).reshape(B, S, -1)
    new_feats = jnp.transpose(new_feats, (0, 2, 1))
    return (new_xyz, new_feats)


# speedup vs baseline: 2.1412x; 2.1412x over previous
import functools

import jax
import jax.numpy as jnp
import numpy as np
from jax.experimental import pallas as pl
from jax.experimental.pallas import tpu as pltpu

N_SAMPLE = 8192
K = 32
IN_CH = 16
MLP_CHANNELS = [16, 32]
EPS = 1e-5

QB = 128


def _knn_body(q_ref, s_ref, sn_ref, o_ref):
    n = s_ref.shape[1]
    d = jnp.dot(q_ref[...], s_ref[...], preferred_element_type=jnp.float32)
    d = d + sn_ref[...]
    iota = jax.lax.broadcasted_iota(jnp.int32, (QB, n), 1)
    big = jnp.int32(n)
    for t in range(K):
        m = jnp.min(d, axis=1, keepdims=True)
        am = jnp.min(jnp.where(d == m, iota, big), axis=1)
        o_ref[:, t] = am
        d = jnp.where(iota == am[:, None], jnp.inf, d)


def _knn(support, query):
    n = support.shape[0]
    s = query.shape[0]
    sn = jnp.sum(support * support, axis=1)[None, :]
    s_aug = jnp.zeros((8, n), jnp.float32)
    s_aug = s_aug.at[0:3, :].set(-2.0 * support.T)
    q_aug = jnp.zeros((s, 8), jnp.float32)
    q_aug = q_aug.at[:, 0:3].set(query)
    return pl.pallas_call(
        _knn_body,
        grid=(s // QB,),
        in_specs=[
            pl.BlockSpec((QB, 8), lambda i: (i, 0)),
            pl.BlockSpec((8, n), lambda i: (0, 0)),
            pl.BlockSpec((1, n), lambda i: (0, 0)),
        ],
        out_specs=pl.BlockSpec((QB, K), lambda i: (i, 0)),
        out_shape=jax.ShapeDtypeStruct((s, K), jnp.int32),
    )(q_aug, s_aug, sn)


def _shared_mlp(x, W, gamma, beta):
    y = jnp.einsum('oi,gik->gok', W, x)
    mean = jnp.mean(y, axis=(0, 2), keepdims=True)
    var = jnp.mean((y - mean) ** 2, axis=(0, 2), keepdims=True)
    y = (y - mean) / jnp.sqrt(var + EPS)
    y = y * gamma[None, :, None] + beta[None, :, None]
    return jax.nn.relu(y)


def kernel(xyz, feats, W1, g1, b1, W2, g2, b2):
    B, C, N = feats.shape
    S = min(N_SAMPLE, N)
    perm = jax.random.permutation(jax.random.key(42), N)[:S]
    new_xyz = xyz[:, :, perm]
    support = xyz[0].T
    query = new_xyz[0].T
    idx = _knn(support, query)
    idx_t = idx.reshape(-1)
    gathered = feats[:, :, idx_t].reshape(B, C, S, K)
    rel_xyz = xyz[:, :, idx_t].reshape(B, 3, S, K) - new_xyz[:, :, :, None]
    group = jnp.concatenate([gathered, rel_xyz], axis=1)
    group = jnp.transpose(group, (0, 2, 1, 3)).reshape(B * S, C + 3, K)
    h = _shared_mlp(group, W1, g1, b1)
    h = _shared_mlp(h, W2, g2, b2)
    new_feats = jnp.max(h, axis=2).reshape(B, S, -1)
    new_feats = jnp.transpose(new_feats, (0, 2, 1))
    return (new_xyz, new_feats)
